# baseline (speedup 1.0000x reference)
import jax
import jax.numpy as jnp
from jax import lax
from jax.experimental import pallas as pl
from jax.experimental.pallas import tpu as pltpu
from jax.experimental.pallas import tpu_sc as plsc

_B = 32
_K = 500
_KPAD = 512
_NCH = 4
_CHUNK = 128
_NC = 2
_NS = 16
_LANES = 16


def _tec_body(flat_hbm, ind_hbm, msk_hbm, tgt_hbm, out_hbm,
              idx_v, val_v, msk_v, tgt_v, acc_v, tot_v, sem):
    cid = lax.axis_index("c")
    sid = lax.axis_index("s")
    b = cid * _NS + sid

    pltpu.sync_copy(ind_hbm.at[b], idx_v)
    pltpu.sync_copy(msk_hbm.at[b], msk_v)
    pltpu.sync_copy(tgt_hbm.at[b], tgt_v)

    hw = flat_hbm.shape[0] // _B
    off = b * hw
    for j in range(_NCH):
        for i in range(_CHUNK // _LANES):
            sl = pl.ds(i * _LANES, _LANES)
            idx_v[j, sl] = idx_v[j, sl] + off

    copies = [
        pltpu.async_copy(flat_hbm.at[idx_v.at[j]], val_v.at[j], sem)
        for j in range(_NCH)
    ]
    for cp in copies:
        cp.wait()

    acc = jnp.zeros((_LANES,), jnp.float32)
    for j in range(_NCH):
        for i in range(_CHUNK // _LANES):
            sl = pl.ds(i * _LANES, _LANES)
            x = val_v[j, sl]
            sig = 1.0 / (1.0 + jnp.exp(-x))
            sig = jnp.minimum(jnp.maximum(sig, 0.0001), 1.0 - 0.0001)
            d = (sig - tgt_v[j, sl]) * msk_v[j, sl]
            acc = acc + d * d
    acc_v[...] = acc

    pltpu.sync_copy(acc_v, out_hbm.at[b])
    plsc.subcore_barrier()

    @pl.when(sid == 0)
    def _():
        pltpu.sync_copy(out_hbm.at[pl.ds(cid * _NS, _NS)], tot_v)
        tot = jnp.zeros((_LANES,), jnp.float32)
        for r in range(_NS):
            tot = tot + tot_v[r, pl.ds(0, _LANES)]
        dnums = lax.GatherDimensionNumbers(
            offset_dims=(), collapsed_slice_dims=(0,), start_index_map=(0,))
        for k in (8, 4, 2, 1):
            perm = lax.iota(jnp.int32, _LANES) ^ k
            shuf = lax.gather(tot, perm[:, None], dnums, slice_sizes=(1,),
                              mode=lax.GatherScatterMode.PROMISE_IN_BOUNDS)
            tot = tot + shuf
        acc_v[...] = tot * (1.0 / float(_B * _K))
        pltpu.sync_copy(acc_v, out_hbm.at[_B + cid])


@jax.jit
def _run(flat, ind3, msk3, tgt3):
    mesh = plsc.VectorSubcoreMesh(core_axis_name="c", subcore_axis_name="s",
                                  num_cores=_NC, num_subcores=_NS)
    return pl.kernel(
        _tec_body,
        out_type=jax.ShapeDtypeStruct((_B + _NC, _LANES), jnp.float32),
        mesh=mesh,
        scratch_types=[
            pltpu.VMEM((_NCH, _CHUNK), jnp.int32),
            pltpu.VMEM((_NCH, _CHUNK), jnp.float32),
            pltpu.VMEM((_NCH, _CHUNK), jnp.float32),
            pltpu.VMEM((_NCH, _CHUNK), jnp.float32),
            pltpu.VMEM((_LANES,), jnp.float32),
            pltpu.VMEM((_NS, _LANES), jnp.float32),
            pltpu.SemaphoreType.DMA,
        ],
    )(flat, ind3, msk3, tgt3)


def kernel(output, mask, ind, target):
    B, C, H, W = output.shape
    flat = jnp.transpose(output, (0, 2, 3, 1)).reshape(B * H * W * C)
    pad = ((0, 0), (0, _KPAD - _K))
    ind3 = jnp.pad(ind, pad).reshape(_B, _NCH, _CHUNK)
    msk3 = jnp.pad(mask.astype(jnp.float32), pad).reshape(_B, _NCH, _CHUNK)
    tgt3 = jnp.pad(target, pad).reshape(_B, _NCH, _CHUNK)
    out = _run(flat, ind3, msk3, tgt3)
    return out[_B, 0] + out[_B + 1, 0]


# speedup vs baseline: 1.0176x; 1.0176x over previous
import jax
import jax.numpy as jnp
from jax import lax
from jax.experimental import pallas as pl
from jax.experimental.pallas import tpu as pltpu
from jax.experimental.pallas import tpu_sc as plsc

_B = 32
_K = 500
_KPAD = 512
_NCH = 4
_CHUNK = 128
_NC = 2
_NS = 16
_LANES = 16


def _tec_body(flat_hbm, ind_hbm, msk_hbm, tgt_hbm, out_hbm,
              idx_v, val_v, msk_v, tgt_v, acc_v, tot_v,
              sem_i, sem_mt, sem_g):
    cid = lax.axis_index("c")
    sid = lax.axis_index("s")
    b = cid * _NS + sid

    cp_i = pltpu.async_copy(ind_hbm.at[b], idx_v, sem_i)
    cp_m = pltpu.async_copy(msk_hbm.at[b], msk_v, sem_mt)
    cp_t = pltpu.async_copy(tgt_hbm.at[b], tgt_v, sem_mt)
    cp_i.wait()

    hw = flat_hbm.shape[0] // _B
    off = b * hw
    for j in range(_NCH):
        for i in range(_CHUNK // _LANES):
            sl = pl.ds(i * _LANES, _LANES)
            idx_v[j, sl] = idx_v[j, sl] + off

    copies = [
        pltpu.async_copy(flat_hbm.at[idx_v.at[j]], val_v.at[j], sem_g)
        for j in range(_NCH)
    ]
    cp_m.wait()
    cp_t.wait()
    for cp in copies:
        cp.wait()

    acc = jnp.zeros((_LANES,), jnp.float32)
    for j in range(_NCH):
        for i in range(_CHUNK // _LANES):
            sl = pl.ds(i * _LANES, _LANES)
            x = val_v[j, sl]
            m = msk_v[j, sl].astype(jnp.float32)
            sig = 1.0 / (1.0 + jnp.exp(-x))
            sig = jnp.minimum(jnp.maximum(sig, 0.0001), 1.0 - 0.0001)
            d = (sig - tgt_v[j, sl]) * m
            acc = acc + d * d
    acc_v[...] = acc

    pltpu.sync_copy(acc_v, out_hbm.at[b])
    plsc.subcore_barrier()

    @pl.when(sid == 0)
    def _():
        pltpu.sync_copy(out_hbm.at[pl.ds(cid * _NS, _NS)], tot_v)
        tot = jnp.zeros((_LANES,), jnp.float32)
        for r in range(_NS):
            tot = tot + tot_v[r, pl.ds(0, _LANES)]
        dnums = lax.GatherDimensionNumbers(
            offset_dims=(), collapsed_slice_dims=(0,), start_index_map=(0,))
        for k in (8, 4, 2, 1):
            perm = lax.iota(jnp.int32, _LANES) ^ k
            shuf = lax.gather(tot, perm[:, None], dnums, slice_sizes=(1,),
                              mode=lax.GatherScatterMode.PROMISE_IN_BOUNDS)
            tot = tot + shuf
        acc_v[...] = tot * (1.0 / float(_B * _K))
        pltpu.sync_copy(acc_v, out_hbm.at[_B + cid])


@jax.jit
def _run(flat, ind3, msk3, tgt3):
    mesh = plsc.VectorSubcoreMesh(core_axis_name="c", subcore_axis_name="s",
                                  num_cores=_NC, num_subcores=_NS)
    return pl.kernel(
        _tec_body,
        out_type=jax.ShapeDtypeStruct((_B + _NC, _LANES), jnp.float32),
        mesh=mesh,
        scratch_types=[
            pltpu.VMEM((_NCH, _CHUNK), jnp.int32),
            pltpu.VMEM((_NCH, _CHUNK), jnp.float32),
            pltpu.VMEM((_NCH, _CHUNK), jnp.int32),
            pltpu.VMEM((_NCH, _CHUNK), jnp.float32),
            pltpu.VMEM((_LANES,), jnp.float32),
            pltpu.VMEM((_NS, _LANES), jnp.float32),
            pltpu.SemaphoreType.DMA,
            pltpu.SemaphoreType.DMA,
            pltpu.SemaphoreType.DMA,
        ],
    )(flat, ind3, msk3, tgt3)


def kernel(output, mask, ind, target):
    B, C, H, W = output.shape
    flat = output.reshape(B * H * W * C)
    pad = ((0, 0), (0, _KPAD - _K))
    ind3 = jnp.pad(ind, pad).reshape(_B, _NCH, _CHUNK)
    msk3 = jnp.pad(mask, pad).reshape(_B, _NCH, _CHUNK)
    tgt3 = jnp.pad(target, pad).reshape(_B, _NCH, _CHUNK)
    out = _run(flat, ind3, msk3, tgt3)
    return out[_B, 0] + out[_B + 1, 0]
